# baseline (speedup 1.0000x reference)
import functools

import jax
import jax.numpy as jnp
from jax import lax
from jax.experimental import pallas as pl
from jax.experimental.pallas import tpu as pltpu
from jax.experimental.pallas import tpu_sc as plsc

B = 16384
D = 16
NC = 2
NS = 16
NW = NC * NS
BPW = B // NW
CHUNK = 128
NCHUNK = BPW // CHUNK
NTILE = BPW // D

_mesh = plsc.VectorSubcoreMesh(core_axis_name="c", subcore_axis_name="s")


def _rsqrt16(x):
    i = lax.bitcast_convert_type(x, jnp.int32)
    i = jnp.int32(0x5F3759DF) - lax.shift_right_arithmetic(i, 1)
    y = lax.bitcast_convert_type(i, jnp.float32)
    for _ in range(3):
        y = y * (jnp.float32(1.5) - jnp.float32(0.5) * x * y * y)
    return y


@functools.partial(
    pl.kernel,
    mesh=_mesh,
    compiler_params=pltpu.CompilerParams(
        needs_layout_passes=False, use_tc_tiling_on_sc=False),
    out_type=jax.ShapeDtypeStruct((B,), jnp.float32),
    scratch_types=[
        pltpu.VMEM((BPW,), jnp.int32),
        pltpu.VMEM((BPW,), jnp.int32),
        pltpu.VMEM((BPW, D), jnp.float32),
        pltpu.VMEM((BPW, D), jnp.float32),
        pltpu.VMEM((BPW, D), jnp.float32),
        pltpu.VMEM((D,), jnp.float32),
        pltpu.VMEM((D,), jnp.float32),
        pltpu.VMEM((BPW,), jnp.float32),
        pltpu.SemaphoreType.DMA,
    ],
)
def _wnom_sc(legs_hbm, votes_hbm, ip_hbm, yp_hbm, np_hbm, w_hbm, beta_hbm,
             out_hbm, lidx, vidx, ra, ry, rn, wv, bv, outv, sem):
    wid = lax.axis_index("s") * NC + lax.axis_index("c")
    base = wid * BPW

    pltpu.sync_copy(legs_hbm.at[pl.ds(base, BPW)], lidx)
    pltpu.sync_copy(votes_hbm.at[pl.ds(base, BPW)], vidx)
    pltpu.sync_copy(w_hbm, wv)
    pltpu.sync_copy(beta_hbm, bv)

    copies = []
    for c in range(NCHUNK):
        sl = pl.ds(c * CHUNK, CHUNK)
        copies.append(pltpu.async_copy(ip_hbm.at[lidx.at[sl]], ra.at[sl], sem))
        copies.append(pltpu.async_copy(yp_hbm.at[vidx.at[sl]], ry.at[sl], sem))
        copies.append(pltpu.async_copy(np_hbm.at[vidx.at[sl]], rn.at[sl], sem))
    for cp in copies:
        cp.wait()

    iota = lax.iota(jnp.int32, 16)
    cols = [jnp.full((16,), d, jnp.int32) for d in range(D)]
    w_vec = wv[...]
    w2_vec = w_vec * w_vec
    w2 = [w2_vec[d] for d in range(D)]
    beta16 = bv[...]
    zeros = jnp.zeros((16,), jnp.float32)

    def tile_body(t, carry):
        rows = t * 16 + iota
        na = ny = nn = zeros
        A = Y = N = C1 = C2 = zeros
        for d in range(D):
            a = plsc.load_gather(ra, [rows, cols[d]])
            y = plsc.load_gather(ry, [rows, cols[d]])
            n = plsc.load_gather(rn, [rows, cols[d]])
            aa = a * a
            yy = y * y
            nn2 = n * n
            na = na + aa
            ny = ny + yy
            nn = nn + nn2
            A = A + aa * w2[d]
            Y = Y + yy * w2[d]
            N = N + nn2 * w2[d]
            C1 = C1 + (a * y) * w2[d]
            C2 = C2 + (a * n) * w2[d]
        one = jnp.float32(1.0)
        sa = jnp.minimum(one, _rsqrt16(na))
        sy = jnp.minimum(one, _rsqrt16(ny))
        sn = jnp.minimum(one, _rsqrt16(nn))
        saA = sa * sa * A
        d1 = saA - jnp.float32(2.0) * (sa * sy) * C1 + sy * sy * Y
        d2 = saA - jnp.float32(2.0) * (sa * sn) * C2 + sn * sn * N
        r = beta16 * (jnp.exp(jnp.float32(-0.5) * d1)
                      - jnp.exp(jnp.float32(-0.5) * d2))
        outv[pl.ds(t * 16, 16)] = r
        return carry

    lax.fori_loop(0, NTILE, tile_body, 0)

    pltpu.sync_copy(outv, out_hbm.at[pl.ds(base, BPW)])


def kernel(legs, votes, ideal_points, yes_points, no_points, w, beta):
    legs32 = legs.astype(jnp.int32)
    votes32 = votes.astype(jnp.int32)
    beta16 = jnp.broadcast_to(beta.astype(jnp.float32), (D,))
    return _wnom_sc(legs32, votes32,
                    ideal_points.astype(jnp.float32),
                    yes_points.astype(jnp.float32),
                    no_points.astype(jnp.float32),
                    w.astype(jnp.float32), beta16)
